# baseline (speedup 1.0000x reference)
import jax
import jax.numpy as jnp
from jax import lax
from jax.experimental import pallas as pl
from jax.experimental.pallas import tpu as pltpu

B = 1024
K = 100000
EPS = 0.1
LAMB = max(5.0, 1500.0 / (1.0 + 0.1 * 1))
COEFF = 1.0 / (1.0 + LAMB)

CHUNK = 2048
NCHUNK = (K + CHUNK - 1) // CHUNK
NSPLIT = 4
STRIPE = CHUNK // NSPLIT
SUBT = STRIPE // 128
ROWS_PER_STEP = (B + NCHUNK - 1) // NCHUNK


def _window_copies(tgt_smem, cos_hbm, phi_hbm, cw_ref, pw_ref, sem_c, sem_p, i):
    t = tgt_smem[i]
    col = pl.multiple_of((t // 128) * 128, 128)
    row8 = pl.multiple_of((i // 8) * 8, 8)
    cp_c = pltpu.make_async_copy(
        cos_hbm.at[pl.ds(row8, 8), pl.ds(col, 128)], cw_ref.at[i], sem_c)
    cp_p = pltpu.make_async_copy(
        phi_hbm.at[pl.ds(row8, 8), pl.ds(col, 128)], pw_ref.at[i], sem_p)
    return cp_c, cp_p


def _tc_body(tgt_smem, cos0, cos1, cos2, cos3, cos_hbm, phi_hbm, tgt_ref,
             out_ref, m_ref, s_ref, r_ref, cw_ref, pw_ref, sem_c, sem_p):
    cos_refs = (cos0, cos1, cos2, cos3)
    c = pl.program_id(0)

    @pl.when(c == 0)
    def _init():
        m_ref[...] = jnp.full((B, 128), -jnp.inf, jnp.float32)
        s_ref[...] = jnp.zeros((B, 128), jnp.float32)
        r_ref[...] = jnp.zeros((B, 128), jnp.float32)

    lo = c * ROWS_PER_STEP
    hi = jnp.minimum(lo + ROWS_PER_STEP, B)

    def _enq(i, carry):
        cp_c, cp_p = _window_copies(tgt_smem, cos_hbm, phi_hbm,
                                    cw_ref, pw_ref, sem_c, sem_p, i)
        cp_c.start()
        cp_p.start()
        return carry

    lax.fori_loop(lo, hi, _enq, 0)

    def _accumulate(masked):
        rem = K - c * CHUNK
        io = lax.broadcasted_iota(jnp.int32, (B, 128), 1)
        cm = m_ref[...]
        for j, ref in enumerate(cos_refs):
            for k in range(SUBT):
                xa = ref[:, k * 128:(k + 1) * 128]
                if masked:
                    xa = jnp.where(io < (rem - j * STRIPE - k * 128), xa,
                                   -jnp.inf)
                cm = jnp.maximum(cm, xa)
        m_old = m_ref[...]
        s = s_ref[...] * jnp.exp(m_old - cm)
        r = r_ref[...]
        for j, ref in enumerate(cos_refs):
            for k in range(SUBT):
                xa = ref[:, k * 128:(k + 1) * 128]
                if masked:
                    valid = io < (rem - j * STRIPE - k * 128)
                    s = s + jnp.exp(jnp.where(valid, xa, -jnp.inf) - cm)
                    r = r + jnp.where(valid, xa, 0.0)
                else:
                    s = s + jnp.exp(xa - cm)
                    r = r + xa
        m_ref[...] = cm
        s_ref[...] = s
        r_ref[...] = r
        return cm, s, r

    @pl.when(c < NCHUNK - 1)
    def _main():
        _accumulate(False)

    @pl.when(c == NCHUNK - 1)
    def _last():
        m_acc, s_acc, r_acc = _accumulate(True)

        def _drain(i, carry):
            cp_c, cp_p = _window_copies(tgt_smem, cos_hbm, phi_hbm,
                                        cw_ref, pw_ref, sem_c, sem_p, i)
            cp_c.wait()
            cp_p.wait()
            return carry

        lax.fori_loop(0, B, _drain, 0)

        m_row = jnp.max(m_acc, axis=1, keepdims=True)
        s_row = jnp.sum(s_acc * jnp.exp(m_acc - m_row), axis=1, keepdims=True)
        r_row = jnp.sum(r_acc, axis=1, keepdims=True)

        tv = tgt_ref[...]
        lane = (tv % 128).reshape(B, 1, 1)
        sub = (lax.broadcasted_iota(jnp.int32, (B, 1), 0) % 8).reshape(B, 1, 1)
        d1 = lax.broadcasted_iota(jnp.int32, (B, 8, 128), 1)
        d2 = lax.broadcasted_iota(jnp.int32, (B, 8, 128), 2)
        sel = jnp.logical_and(d1 == sub, d2 == lane)
        ct = jnp.sum(jnp.where(sel, cw_ref[...], 0.0), axis=(1, 2)).reshape(B, 1)
        pt = jnp.sum(jnp.where(sel, pw_ref[...], 0.0), axis=(1, 2)).reshape(B, 1)

        delta = (pt - ct) * COEFF
        ot = ct + delta
        m2 = jnp.maximum(m_row, ot)
        s2 = (s_row * jnp.exp(m_row - m2)
              + jnp.exp(ot - m2) - jnp.exp(ct - m2))
        lse = m2 + jnp.log(s2)
        per_row = ((1.0 - EPS) * (ot - lse)
                   + (EPS / K) * ((r_row + delta) - K * lse))
        out_ref[...] = -jnp.sum(per_row, keepdims=True) / B


_tc_loss = pl.pallas_call(
    _tc_body,
    grid_spec=pltpu.PrefetchScalarGridSpec(
        num_scalar_prefetch=1,
        grid=(NCHUNK,),
        in_specs=[
            pl.BlockSpec((B, STRIPE), lambda c, tgt: (0, c * NSPLIT)),
            pl.BlockSpec((B, STRIPE), lambda c, tgt: (0, c * NSPLIT + 1)),
            pl.BlockSpec((B, STRIPE), lambda c, tgt: (0, c * NSPLIT + 2)),
            pl.BlockSpec((B, STRIPE), lambda c, tgt: (0, c * NSPLIT + 3)),
            pl.BlockSpec(memory_space=pltpu.HBM),
            pl.BlockSpec(memory_space=pltpu.HBM),
            pl.BlockSpec((B, 1), lambda c, tgt: (0, 0)),
        ],
        out_specs=pl.BlockSpec((1, 1), lambda c, tgt: (0, 0)),
        scratch_shapes=[
            pltpu.VMEM((B, 128), jnp.float32),
            pltpu.VMEM((B, 128), jnp.float32),
            pltpu.VMEM((B, 128), jnp.float32),
            pltpu.VMEM((B, 8, 128), jnp.float32),
            pltpu.VMEM((B, 8, 128), jnp.float32),
            pltpu.SemaphoreType.DMA,
            pltpu.SemaphoreType.DMA,
        ],
    ),
    out_shape=jax.ShapeDtypeStruct((1, 1), jnp.float32),
)


def kernel(cos_theta, phi_theta, targets):
    loss = _tc_loss(targets, cos_theta, cos_theta, cos_theta, cos_theta,
                    cos_theta, phi_theta, targets.reshape(B, 1))
    return loss[0, 0]
